# baseline (speedup 1.0000x reference)
import jax
import jax.numpy as jnp
from jax import lax
from jax.experimental import pallas as pl
from jax.experimental.pallas import tpu as pltpu
from jax.experimental.pallas import tpu_sc as plsc

B = 16384
NS = 26
ND = 13
V = 100000
VB = 782
D = 4
L = 16
NW = 32
RPW = B // NW
CH = 128
NB = RPW // CH
NCH = NB * NS * D
IPB = NS * D * CH
NT = CH // L


def _fm_body(xd_hbm, wb_hbm, idx_hbm, tab_hbm, out_hbm,
             idx_v, data_v, xd_v, w_v, out_v, sem0, sem1):
    wid = lax.axis_index("s") * 2 + lax.axis_index("c")
    base = wid * RPW

    pltpu.sync_copy(idx_hbm.at[wid], idx_v)
    pltpu.sync_copy(xd_hbm.at[wid], xd_v)
    pltpu.sync_copy(wb_hbm, w_v)

    wreg = [w_v[pl.ds(j * L, L)] for j in range(ND)]
    sems = [sem0, sem1]

    def fire(b):
        po = (b & 1) * IPB
        for c in range(NS * D):
            pltpu.async_copy(
                tab_hbm.at[idx_v.at[pl.ds((b * NS * D + c) * CH, CH)]],
                data_v.at[pl.ds(po + c * CH, CH)], sems[b & 1])

    def drain(b):
        pltpu.make_async_copy(tab_hbm.at[pl.ds(0, IPB)],
                              data_v.at[pl.ds((b & 1) * IPB, IPB)],
                              sems[b & 1]).wait()

    fire(0)
    for b in range(NB):
        if b + 1 < NB:
            fire(b + 1)
        drain(b)
        po = (b & 1) * IPB

        def t_body(T, _, b=b, po=po):
            row0 = T * L
            s16 = data_v[pl.ds(po, L)] * 0.0
            q16 = s16
            for j in range(ND):
                x = xd_v[pl.ds(j * RPW + b * CH + row0, L)]
                t = x * wreg[j]
                s16 = s16 + t
                q16 = q16 + t * t

            gd = [s16 * 0.0 for _ in range(D)]
            qd = [s16 * 0.0 for _ in range(D)]
            for i in range(NS):
                for d in range(D):
                    v = data_v[pl.ds(po + (i * D + d) * CH + row0, L)]
                    gd[d] = gd[d] + v
                    qd[d] = qd[d] + v * v

            a16 = gd[0] * gd[0]
            b16 = gd[0]
            c16 = qd[0]
            for d in range(1, D):
                a16 = a16 + gd[d] * gd[d]
                b16 = b16 + gd[d]
                c16 = c16 + qd[d]
            o = 0.5 * (a16 + 2.0 * s16 * b16 + 4.0 * s16 * s16
                       - c16 - 4.0 * q16)
            out_v[pl.ds(b * CH + row0, L)] = o
            return ()

        lax.fori_loop(0, NT, t_body, ())

    pltpu.sync_copy(out_v, out_hbm.at[pl.ds(base, RPW)])


@jax.jit
def _fm_sc(xd, wb, idx4, tab):
    mesh = plsc.VectorSubcoreMesh(core_axis_name="c", subcore_axis_name="s")
    return pl.kernel(
        _fm_body,
        out_type=jax.ShapeDtypeStruct((B,), jnp.float32),
        mesh=mesh,
        scratch_types=[
            pltpu.VMEM((NB * IPB,), jnp.int32),
            pltpu.VMEM((2 * IPB,), jnp.float32),
            pltpu.VMEM((ND * RPW,), jnp.float32),
            pltpu.VMEM((ND * L,), jnp.float32),
            pltpu.VMEM((RPW,), jnp.float32),
            pltpu.SemaphoreType.DMA,
            pltpu.SemaphoreType.DMA,
        ],
        compiler_params=pltpu.CompilerParams(use_tc_tiling_on_sc=False),
    )(xd, wb, idx4, tab)


def kernel(X_dense, tables, weight, X_sparse):
    tpad = jnp.pad(tables, ((0, 0), (0, VB * 128 - V), (0, 0)))
    tab = (tpad.transpose(0, 2, 1).reshape(NS, D, VB, 128)
           .transpose(0, 2, 1, 3).reshape(NS, VB * D, 128)
           .reshape(NS * VB * D * 128))

    vv = X_sparse.reshape(NW, NB, CH, NS)
    a = ((vv >> 7) * (D * 128) + (vv & 127)
         + (jnp.arange(NS, dtype=jnp.int32) * (VB * D * 128))[None, None, None, :])
    idx4 = (a.transpose(0, 1, 3, 2)[:, :, :, None, :]
            + (jnp.arange(D, dtype=jnp.int32) * 128)[None, None, None, :, None]
            ).reshape(NW, NB * IPB)

    xd = X_dense.T.reshape(ND, NW, RPW).transpose(1, 0, 2).reshape(NW, ND * RPW)
    wb = jnp.broadcast_to(weight.reshape(ND, 1), (ND, L)).reshape(ND * L)
    out = _fm_sc(xd, wb, idx4, tab)
    return out.reshape(B, 1)


# speedup vs baseline: 1.0437x; 1.0437x over previous
import jax
import jax.numpy as jnp
from jax import lax
from jax.experimental import pallas as pl
from jax.experimental.pallas import tpu as pltpu
from jax.experimental.pallas import tpu_sc as plsc

B = 16384
NS = 26
ND = 13
V = 100000
VB = 782
D = 4
L = 16
NW = 32
RPW = B // NW
CH = 128
NB = RPW // CH
NCH = NB * NS * D
IPB = NS * D * CH
NT = CH // L


def _fm_body(xd_hbm, wb_hbm, idx_hbm, tab_hbm, out_hbm,
             idx_v, data_v, xd_v, w_v, out_v, sem0, sem1):
    wid = lax.axis_index("s") * 2 + lax.axis_index("c")
    base = wid * RPW

    pltpu.sync_copy(idx_hbm.at[wid], idx_v)
    pltpu.sync_copy(xd_hbm.at[wid], xd_v)
    pltpu.sync_copy(wb_hbm, w_v)

    wreg = [w_v[pl.ds(j * L, L)] for j in range(ND)]
    sems = [sem0, sem1]

    def fire(b):
        po = (b & 1) * IPB
        for c in range(NS * D):
            pltpu.async_copy(
                tab_hbm.at[idx_v.at[pl.ds((b * NS * D + c) * CH, CH)]],
                data_v.at[pl.ds(po + c * CH, CH)], sems[b & 1])

    def drain(b):
        pltpu.make_async_copy(tab_hbm.at[pl.ds(0, IPB)],
                              data_v.at[pl.ds((b & 1) * IPB, IPB)],
                              sems[b & 1]).wait()

    fire(0)
    for b in range(NB):
        if b + 1 < NB:
            fire(b + 1)
        drain(b)
        po = (b & 1) * IPB

        def t_body(T, _, b=b, po=po):
            row0 = T * L
            s16 = data_v[pl.ds(po, L)] * 0.0
            q16 = s16
            for j in range(ND):
                x = xd_v[pl.ds(j * RPW + b * CH + row0, L)]
                t = x * wreg[j]
                s16 = s16 + t
                q16 = q16 + t * t

            gd = [s16 * 0.0 for _ in range(D)]
            qd = [s16 * 0.0 for _ in range(D)]
            for i in range(NS):
                for d in range(D):
                    v = data_v[pl.ds(po + (i * D + d) * CH + row0, L)]
                    gd[d] = gd[d] + v
                    qd[d] = qd[d] + v * v

            a16 = gd[0] * gd[0]
            b16 = gd[0]
            c16 = qd[0]
            for d in range(1, D):
                a16 = a16 + gd[d] * gd[d]
                b16 = b16 + gd[d]
                c16 = c16 + qd[d]
            o = 0.5 * (a16 + 2.0 * s16 * b16 + 4.0 * s16 * s16
                       - c16 - 4.0 * q16)
            out_v[pl.ds(b * CH + row0, L)] = o
            return ()

        lax.fori_loop(0, NT, t_body, ())

    pltpu.sync_copy(out_v, out_hbm.at[pl.ds(base, RPW)])


@jax.jit
def _fm_sc(xd, wb, idx4, tab):
    mesh = plsc.VectorSubcoreMesh(core_axis_name="c", subcore_axis_name="s")
    return pl.kernel(
        _fm_body,
        out_type=jax.ShapeDtypeStruct((B,), jnp.float32),
        mesh=mesh,
        scratch_types=[
            pltpu.VMEM((NB * IPB,), jnp.int32),
            pltpu.VMEM((2 * IPB,), jnp.float32),
            pltpu.VMEM((ND * RPW,), jnp.float32),
            pltpu.VMEM((ND * L,), jnp.float32),
            pltpu.VMEM((RPW,), jnp.float32),
            pltpu.SemaphoreType.DMA,
            pltpu.SemaphoreType.DMA,
        ],
        compiler_params=pltpu.CompilerParams(use_tc_tiling_on_sc=False),
    )(xd, wb, idx4, tab)


def kernel(X_dense, tables, weight, X_sparse):
    tpad = jnp.pad(tables, ((0, 0), (0, VB * 128 - V), (0, 0)))
    tab = jax.lax.optimization_barrier(
        tpad.transpose(0, 2, 1).reshape(NS, D, VB, 128)
        .transpose(0, 2, 1, 3).reshape(NS, VB * D, 128)
    ).reshape(NS * VB * D * 128)

    vv = X_sparse.reshape(NW, NB, CH, NS)
    a = ((vv >> 7) * (D * 128) + (vv & 127)
         + (jnp.arange(NS, dtype=jnp.int32) * (VB * D * 128))[None, None, None, :])
    idx4 = (a.transpose(0, 1, 3, 2)[:, :, :, None, :]
            + (jnp.arange(D, dtype=jnp.int32) * 128)[None, None, None, :, None]
            ).reshape(NW, NB * IPB)

    xd = X_dense.T.reshape(ND, NW, RPW).transpose(1, 0, 2).reshape(NW, ND * RPW)
    wb = jnp.broadcast_to(weight.reshape(ND, 1), (ND, L)).reshape(ND * L)
    out = _fm_sc(xd, wb, idx4, tab)
    return out.reshape(B, 1)


# speedup vs baseline: 1.0536x; 1.0096x over previous
import jax
import jax.numpy as jnp
from jax import lax
from jax.experimental import pallas as pl
from jax.experimental.pallas import tpu as pltpu
from jax.experimental.pallas import tpu_sc as plsc

B = 16384
NS = 26
ND = 13
V = 100000
VB = 782
D = 4
L = 16
NW = 32
RPW = B // NW
CH = 128
NB = RPW // CH
NCH = NB * NS * D
IPB = NS * D * CH
NT = CH // L


def _fm_body(xd_hbm, wb_hbm, idx_hbm, tab_hbm, out_hbm,
             idx_v, data_v, xd_v, w_v, out_v, sem0, sem1, sem2, sem3):
    wid = lax.axis_index("s") * 2 + lax.axis_index("c")
    base = wid * RPW

    pltpu.sync_copy(idx_hbm.at[wid], idx_v)
    pltpu.sync_copy(xd_hbm.at[wid], xd_v)
    pltpu.sync_copy(wb_hbm, w_v)

    wreg = [w_v[pl.ds(j * L, L)] for j in range(ND)]
    sems = [sem0, sem1, sem2, sem3]

    for b in range(NB):
        for c in range(NS * D):
            pltpu.async_copy(
                tab_hbm.at[idx_v.at[pl.ds((b * NS * D + c) * CH, CH)]],
                data_v.at[pl.ds(b * IPB + c * CH, CH)], sems[b])

    for b in range(NB):
        pltpu.make_async_copy(tab_hbm.at[pl.ds(0, IPB)],
                              data_v.at[pl.ds(b * IPB, IPB)],
                              sems[b]).wait()
        po = b * IPB

        def t_body(T, _, b=b, po=po):
            row0 = T * L
            s16 = data_v[pl.ds(po, L)] * 0.0
            q16 = s16
            for j in range(ND):
                x = xd_v[pl.ds(j * RPW + b * CH + row0, L)]
                t = x * wreg[j]
                s16 = s16 + t
                q16 = q16 + t * t

            gd = [s16 * 0.0 for _ in range(D)]
            qd = [s16 * 0.0 for _ in range(D)]
            for i in range(NS):
                for d in range(D):
                    v = data_v[pl.ds(po + (i * D + d) * CH + row0, L)]
                    gd[d] = gd[d] + v
                    qd[d] = qd[d] + v * v

            a16 = gd[0] * gd[0]
            b16 = gd[0]
            c16 = qd[0]
            for d in range(1, D):
                a16 = a16 + gd[d] * gd[d]
                b16 = b16 + gd[d]
                c16 = c16 + qd[d]
            o = 0.5 * (a16 + 2.0 * s16 * b16 + 4.0 * s16 * s16
                       - c16 - 4.0 * q16)
            out_v[pl.ds(b * CH + row0, L)] = o
            return ()

        lax.fori_loop(0, NT, t_body, ())

    pltpu.sync_copy(out_v, out_hbm.at[pl.ds(base, RPW)])


@jax.jit
def _fm_sc(xd, wb, idx4, tab):
    mesh = plsc.VectorSubcoreMesh(core_axis_name="c", subcore_axis_name="s")
    return pl.kernel(
        _fm_body,
        out_type=jax.ShapeDtypeStruct((B,), jnp.float32),
        mesh=mesh,
        scratch_types=[
            pltpu.VMEM((NB * IPB,), jnp.int32),
            pltpu.VMEM((NB * IPB,), jnp.float32),
            pltpu.VMEM((ND * RPW,), jnp.float32),
            pltpu.VMEM((ND * L,), jnp.float32),
            pltpu.VMEM((RPW,), jnp.float32),
            pltpu.SemaphoreType.DMA,
            pltpu.SemaphoreType.DMA,
            pltpu.SemaphoreType.DMA,
            pltpu.SemaphoreType.DMA,
        ],
        compiler_params=pltpu.CompilerParams(use_tc_tiling_on_sc=False),
    )(xd, wb, idx4, tab)


def kernel(X_dense, tables, weight, X_sparse):
    rzf = weight[0, 0] * 0.0
    tab = (jnp.transpose(tables, (0, 2, 1)).reshape(NS * D * V) + rzf)

    vv = X_sparse.reshape(NW, NB, CH, NS)
    a = vv + (jnp.arange(NS, dtype=jnp.int32) * (D * V))[None, None, None, :]
    idx4 = (a.transpose(0, 1, 3, 2)[:, :, :, None, :]
            + (jnp.arange(D, dtype=jnp.int32) * V)[None, None, None, :, None]
            ).reshape(NW, NB * IPB)

    xd = X_dense.T.reshape(ND, NW, RPW).transpose(1, 0, 2).reshape(NW, ND * RPW)
    wb = jnp.broadcast_to(weight.reshape(ND, 1), (ND, L)).reshape(ND * L)
    out = _fm_sc(xd, wb, idx4, tab)
    return out.reshape(B, 1)


# speedup vs baseline: 1.2198x; 1.1577x over previous
import jax
import jax.numpy as jnp
from jax import lax
from jax.experimental import pallas as pl
from jax.experimental.pallas import tpu as pltpu
from jax.experimental.pallas import tpu_sc as plsc

B = 16384
NS = 26
ND = 13
V = 100000
VB = 782
D = 4
L = 16
NW = 32
RPW = B // NW
CH = 128
NB = RPW // CH
NCH = NB * NS * D
IPB = NS * D * CH
NT = CH // L


def _fm_body(xd_hbm, wb_hbm, idx_hbm, tab_hbm, out_hbm,
             idx_v, data_v, xd_v, w_v, out_v, sem0, sem1, sem2, sem3):
    wid = lax.axis_index("s") * 2 + lax.axis_index("c")
    base = wid * RPW

    pltpu.sync_copy(idx_hbm.at[wid], idx_v)
    pltpu.sync_copy(xd_hbm.at[wid], xd_v)
    pltpu.sync_copy(wb_hbm, w_v)

    wreg = [w_v[pl.ds(j * L, L)] for j in range(ND)]
    sems = [sem0, sem1, sem2, sem3]

    for b in range(NB):
        for c in range(NS * D):
            pltpu.async_copy(
                tab_hbm.at[idx_v.at[pl.ds((b * NS * D + c) * CH, CH)]],
                data_v.at[pl.ds(b * IPB + c * CH, CH)], sems[b])

    for b in range(NB):
        pltpu.make_async_copy(tab_hbm.at[pl.ds(0, IPB)],
                              data_v.at[pl.ds(b * IPB, IPB)],
                              sems[b]).wait()
        po = b * IPB

        def t_body(T, _, b=b, po=po):
            row0 = T * L
            s16 = data_v[pl.ds(po, L)] * 0.0
            q16 = s16
            for j in range(ND):
                x = xd_v[pl.ds(j * RPW + b * CH + row0, L)]
                t = x * wreg[j]
                s16 = s16 + t
                q16 = q16 + t * t

            gd = [s16 * 0.0 for _ in range(D)]
            qd = [s16 * 0.0 for _ in range(D)]
            for i in range(NS):
                for d in range(D):
                    v = data_v[pl.ds(po + (i * D + d) * CH + row0, L)]
                    gd[d] = gd[d] + v
                    qd[d] = qd[d] + v * v

            a16 = gd[0] * gd[0]
            b16 = gd[0]
            c16 = qd[0]
            for d in range(1, D):
                a16 = a16 + gd[d] * gd[d]
                b16 = b16 + gd[d]
                c16 = c16 + qd[d]
            o = 0.5 * (a16 + 2.0 * s16 * b16 + 4.0 * s16 * s16
                       - c16 - 4.0 * q16)
            out_v[pl.ds(b * CH + row0, L)] = o
            return ()

        lax.fori_loop(0, NT, t_body, ())

    pltpu.sync_copy(out_v, out_hbm.at[pl.ds(base, RPW)])


@jax.jit
def _fm_sc(xd, wb, idx4, tab):
    mesh = plsc.VectorSubcoreMesh(core_axis_name="c", subcore_axis_name="s")
    return pl.kernel(
        _fm_body,
        out_type=jax.ShapeDtypeStruct((B,), jnp.float32),
        mesh=mesh,
        scratch_types=[
            pltpu.VMEM((NB * IPB,), jnp.int32),
            pltpu.VMEM((NB * IPB,), jnp.float32),
            pltpu.VMEM((ND * RPW,), jnp.float32),
            pltpu.VMEM((ND * L,), jnp.float32),
            pltpu.VMEM((RPW,), jnp.float32),
            pltpu.SemaphoreType.DMA,
            pltpu.SemaphoreType.DMA,
            pltpu.SemaphoreType.DMA,
            pltpu.SemaphoreType.DMA,
        ],
        compiler_params=pltpu.CompilerParams(use_tc_tiling_on_sc=False),
    )(xd, wb, idx4, tab)


def kernel(X_dense, tables, weight, X_sparse):
    tab = jnp.transpose(tables, (0, 2, 1)).reshape(NS * D * V)

    vv = X_sparse.reshape(NW, NB, CH, NS)
    a = vv + (jnp.arange(NS, dtype=jnp.int32) * (D * V))[None, None, None, :]
    idx4 = (a.transpose(0, 1, 3, 2)[:, :, :, None, :]
            + (jnp.arange(D, dtype=jnp.int32) * V)[None, None, None, :, None]
            ).reshape(NW, NB * IPB)

    xd = X_dense.T.reshape(ND, NW, RPW).transpose(1, 0, 2).reshape(NW, ND * RPW)
    wb = jnp.broadcast_to(weight.reshape(ND, 1), (ND, L)).reshape(ND * L)
    out = _fm_sc(xd, wb, idx4, tab)
    return out.reshape(B, 1)


# speedup vs baseline: 1.3676x; 1.1212x over previous
import jax
import jax.numpy as jnp
from jax import lax
from jax.experimental import pallas as pl
from jax.experimental.pallas import tpu as pltpu
from jax.experimental.pallas import tpu_sc as plsc

B = 16384
NS = 26
ND = 13
V = 100000
VB = 782
D = 4
L = 16
NW = 32
RPW = B // NW
CH = 128
NB = RPW // CH
NCH = NB * NS * D
IPB = NS * D * CH
NT = CH // L


def _fm_body(xd_hbm, wb_hbm, idx_hbm, tab_hbm, out_hbm,
             idx0_v, idx_v, data_v, xd_v, w_v, out_v, sem0, sem1, sem2, sem3):
    wid = lax.axis_index("s") * 2 + lax.axis_index("c")
    base = wid * RPW

    pltpu.sync_copy(idx_hbm.at[wid], idx0_v)
    pltpu.sync_copy(xd_hbm.at[wid], xd_v)
    pltpu.sync_copy(wb_hbm, w_v)

    wreg = [w_v[pl.ds(j * L, L)] for j in range(ND)]
    sems = [sem0, sem1, sem2, sem3]

    def derive(b):
        po = (b & 1) * IPB
        def d_body(n, _):
            v = idx0_v[pl.ds(b * (NS * CH) + n * L, L)]
            c = n // (CH // L)
            k = (n % (CH // L)) * L
            for d in range(D):
                idx_v[pl.ds(po + (c * D + d) * CH + k, L)] = v + d * V
            return ()
        lax.fori_loop(0, NS * CH // L, d_body, ())

    def fire(b):
        po = (b & 1) * IPB
        for c in range(NS * D):
            pltpu.async_copy(
                tab_hbm.at[idx_v.at[pl.ds(po + c * CH, CH)]],
                data_v.at[pl.ds(po + c * CH, CH)], sems[b])

    derive(0)
    fire(0)
    for b in range(NB):
        if b + 1 < NB:
            derive(b + 1)
            fire(b + 1)
        pltpu.make_async_copy(tab_hbm.at[pl.ds(0, IPB)],
                              data_v.at[pl.ds((b & 1) * IPB, IPB)],
                              sems[b]).wait()
        po = (b & 1) * IPB

        def t_body(T, _, b=b, po=po):
            row0 = T * L
            s16 = data_v[pl.ds(po, L)] * 0.0
            q16 = s16
            for j in range(ND):
                x = xd_v[pl.ds(j * RPW + b * CH + row0, L)]
                t = x * wreg[j]
                s16 = s16 + t
                q16 = q16 + t * t

            gd = [s16 * 0.0 for _ in range(D)]
            qd = [s16 * 0.0 for _ in range(D)]
            for i in range(NS):
                for d in range(D):
                    v = data_v[pl.ds(po + (i * D + d) * CH + row0, L)]
                    gd[d] = gd[d] + v
                    qd[d] = qd[d] + v * v

            a16 = gd[0] * gd[0]
            b16 = gd[0]
            c16 = qd[0]
            for d in range(1, D):
                a16 = a16 + gd[d] * gd[d]
                b16 = b16 + gd[d]
                c16 = c16 + qd[d]
            o = 0.5 * (a16 + 2.0 * s16 * b16 + 4.0 * s16 * s16
                       - c16 - 4.0 * q16)
            out_v[pl.ds(b * CH + row0, L)] = o
            return ()

        lax.fori_loop(0, NT, t_body, ())

    pltpu.sync_copy(out_v, out_hbm.at[pl.ds(base, RPW)])


@jax.jit
def _fm_sc(xd, wb, idx4, tab):
    mesh = plsc.VectorSubcoreMesh(core_axis_name="c", subcore_axis_name="s")
    return pl.kernel(
        _fm_body,
        out_type=jax.ShapeDtypeStruct((B,), jnp.float32),
        mesh=mesh,
        scratch_types=[
            pltpu.VMEM((NB * NS * CH,), jnp.int32),
            pltpu.VMEM((2 * IPB,), jnp.int32),
            pltpu.VMEM((2 * IPB,), jnp.float32),
            pltpu.VMEM((ND * RPW,), jnp.float32),
            pltpu.VMEM((ND * L,), jnp.float32),
            pltpu.VMEM((RPW,), jnp.float32),
            pltpu.SemaphoreType.DMA,
            pltpu.SemaphoreType.DMA,
            pltpu.SemaphoreType.DMA,
            pltpu.SemaphoreType.DMA,
        ],
        compiler_params=pltpu.CompilerParams(use_tc_tiling_on_sc=False),
    )(xd, wb, idx4, tab)


def kernel(X_dense, tables, weight, X_sparse):
    tab = jnp.transpose(tables, (0, 2, 1)).reshape(NS * D * V)

    vv = X_sparse.reshape(NW, NB, CH, NS)
    a = vv + (jnp.arange(NS, dtype=jnp.int32) * (D * V))[None, None, None, :]
    idx4 = a.transpose(0, 1, 3, 2).reshape(NW, NB * NS * CH)

    xd = X_dense.T.reshape(ND, NW, RPW).transpose(1, 0, 2).reshape(NW, ND * RPW)
    wb = jnp.broadcast_to(weight.reshape(ND, 1), (ND, L)).reshape(ND * L)
    out = _fm_sc(xd, wb, idx4, tab)
    return out.reshape(B, 1)
